# baseline (speedup 1.0000x reference)
import functools

import jax
import jax.numpy as jnp
from jax import lax
from jax.experimental import pallas as pl
from jax.experimental.pallas import tpu as pltpu
from jax.experimental.pallas import tpu_sc as plsc

BATCH_BLK = 2048


def _encode_body(x_ref, w_ref, b_ref, a1t_ref):
    xblk = x_ref[...]
    z = lax.dot_general(
        w_ref[...], xblk, (((1,), (1,)), ((), ())),
        preferred_element_type=jnp.float32,
        precision=lax.Precision.DEFAULT,
    )
    z = z + b_ref[...]
    a1t_ref[...] = jnp.where(z > 0, z, 0.0)


def _encode(x, W, b2d):
    B, D = x.shape
    U = W.shape[0]
    grid = B // BATCH_BLK
    return pl.pallas_call(
        _encode_body,
        grid=(grid,),
        in_specs=[
            pl.BlockSpec((BATCH_BLK, D), lambda i: (i, 0)),
            pl.BlockSpec((U, D), lambda i: (0, 0)),
            pl.BlockSpec((U, 1), lambda i: (0, 0)),
        ],
        out_specs=pl.BlockSpec((U, BATCH_BLK), lambda i: (0, i)),
        out_shape=jax.ShapeDtypeStruct((U, B), jnp.float32),
    )(x, W, b2d)


def _select_body(kcount, a1t_ref, thr_ref):
    a = a1t_ref[...]
    U = a.shape[0]
    kf = jnp.float32(kcount)

    def step(i, t):
        bit = 30 - i
        cand = t | (jnp.int32(1) << bit)
        candf = lax.bitcast_convert_type(cand, jnp.float32)
        cnt = jnp.sum(jnp.where(a >= candf, 1.0, 0.0), axis=1, keepdims=True)
        return jnp.where(cnt >= kf, cand, t)

    t0 = jnp.zeros((U, 1), jnp.int32)
    t = lax.fori_loop(0, 31, step, t0)
    thr_ref[...] = lax.bitcast_convert_type(t, jnp.float32)


def _select(a1T, kcount):
    U = a1T.shape[0]
    return pl.pallas_call(
        functools.partial(_select_body, kcount),
        out_shape=jax.ShapeDtypeStruct((U, 1), jnp.float32),
    )(a1T)


_SUB_PASSES = ((16, 6), (10, 6), (4, 6), (0, 4))
_P1_SHIFT, _P1_BITS = 22, 10
_P1_NB = 1 << (_P1_BITS - 1)
_P1_STRIDE = _P1_NB + 1
_HIST_WORDS = 16 * _P1_STRIDE + 112


def _sc_select_body(kcount, a1t_hbm, out_hbm, colbuf, cand1, cand2, hist,
                    thrbuf):
    info = plsc.get_sparse_core_info()
    nc = info.num_cores
    wid = lax.axis_index("s") * nc + lax.axis_index("c")
    lanes = lax.iota(jnp.int32, 16)
    ones = jnp.ones((16,), jnp.int32)
    kc = jnp.int32(kcount)

    def zero_hist(nwords):
        def zb(i, _):
            for t in range(8):
                hist[pl.ds((i * 8 + t) * 16, 16)] = jnp.zeros((16,), jnp.int32)
            return 0
        lax.fori_loop(0, (nwords + 127) // 128, zb, 0)

    def find_bucket(nb, stride, base_list, r):
        above = jnp.int32(0)
        bstar = jnp.int32(-1)
        rnew = jnp.int32(-1)
        for g in reversed(base_list):
            acc = jnp.zeros((16,), jnp.int32)
            for l in range(16):
                acc = acc + hist[pl.ds(l * stride + g, 16)]
            s = lax.rev(jnp.cumsum(lax.rev(acc, (0,))), (0,))
            si = s + above
            se = si - acc
            m = (se < r) & (si >= r)
            bcand = jnp.where(m, g + lanes, jnp.int32(-1))
            bmax = lax.reduce_max(bcand, (0,))
            semax = lax.reduce_max(jnp.where(m, se, jnp.int32(0)), (0,))
            found = bmax >= 0
            bstar = jnp.where(found, bmax, bstar)
            rnew = jnp.where(found, r - semax, rnew)
            above = above + lax.reduce_sum(acc, (0,))
        return bstar, rnew

    def find_bucket_dyn(stride, ngroups, r):
        def grp(gi, carry):
            above, bstar, rnew = carry
            base = (jnp.int32(ngroups - 1) - gi) * 16
            acc = jnp.zeros((16,), jnp.int32)
            for l in range(16):
                acc = acc + hist[pl.ds(l * stride + base, 16)]
            s = lax.rev(jnp.cumsum(lax.rev(acc, (0,))), (0,))
            si = s + above
            se = si - acc
            m = (se < r) & (si >= r)
            bcand = jnp.where(m, base + lanes, jnp.int32(-1))
            bmax = lax.reduce_max(bcand, (0,))
            semax = lax.reduce_max(jnp.where(m, se, jnp.int32(0)), (0,))
            found = bmax >= 0
            bstar = jnp.where(found, bmax, bstar)
            rnew = jnp.where(found, r - semax, rnew)
            above = above + lax.reduce_sum(acc, (0,))
            return above, bstar, rnew
        _, bstar, rnew = lax.fori_loop(
            0, ngroups, grp, (jnp.int32(0), jnp.int32(-1), jnp.int32(-1)))
        return bstar, rnew

    def process_column(j, thrvec):
        col = wid * jnp.int32(_COLS_PER_TILE) + j
        pltpu.sync_copy(a1t_hbm.at[col], colbuf)
        nvec = colbuf.shape[0] // 16

        zero_hist(16 * _P1_STRIDE)
        lane_base1 = lanes * jnp.int32(_P1_STRIDE)

        def h1(i, _):
            for t in range(4):
                v = colbuf[pl.ds((i * 4 + t) * 16, 16)]
                u = lax.bitcast_convert_type(v, jnp.int32)
                bkt = lax.shift_right_logical(u, jnp.int32(_P1_SHIFT))
                plsc.addupdate_scatter(hist, [lane_base1 + bkt], ones)
            return 0
        lax.fori_loop(0, nvec // 4, h1, 0)
        b1, r = find_bucket_dyn(_P1_STRIDE, _P1_NB // 16, kc)

        def cp1(i, cnt):
            for t in range(4):
                v = colbuf[pl.ds((i * 4 + t) * 16, 16)]
                u = lax.bitcast_convert_type(v, jnp.int32)
                m = lax.shift_right_logical(u, jnp.int32(_P1_SHIFT)) == b1
                pc = jnp.cumsum(m.astype(jnp.int32))
                pos = cnt + pc - 1
                plsc.store_scatter(cand1, [pos], u, mask=m)
                cnt = cnt + lax.reduce_max(pc, (0,))
            return cnt
        n = lax.fori_loop(0, nvec // 4, cp1, jnp.int32(0))
        prefix = b1
        src, dst = cand1, cand2

        for pi, (shift, bits) in enumerate(_SUB_PASSES):
            nb = 1 << bits
            stride = nb + 1
            zero_hist(16 * stride)
            lane_base = lanes * jnp.int32(stride)
            shift_c = jnp.int32(shift)
            mask_c = jnp.int32(nb - 1)
            nvr = lax.shift_right_logical(n + jnp.int32(15), 4)

            def hb(i, _):
                base_i = i * 16
                u = src[pl.ds(base_i, 16)]
                valid = (base_i + lanes) < n
                bkt = lax.shift_right_logical(u, shift_c) & mask_c
                plsc.addupdate_scatter(hist, [lane_base + bkt], ones,
                                       mask=valid)
                return 0
            lax.fori_loop(0, nvr, hb, 0)
            bstar, r = find_bucket(nb, stride,
                                   [g * 16 for g in range(nb // 16)], r)
            prefix = lax.shift_left(prefix, jnp.int32(bits)) | bstar

            if pi != len(_SUB_PASSES) - 1:
                def cpb(i, cnt):
                    base_i = i * 16
                    u = src[pl.ds(base_i, 16)]
                    valid = (base_i + lanes) < n
                    m = (lax.shift_right_logical(u, shift_c) == prefix) & valid
                    pc = jnp.cumsum(m.astype(jnp.int32))
                    pos = cnt + pc - 1
                    plsc.store_scatter(dst, [pos], u, mask=m)
                    return cnt + lax.reduce_max(pc, (0,))
                n = lax.fori_loop(0, nvr, cpb, jnp.int32(0))
                src, dst = dst, src

        thrf = lax.bitcast_convert_type(jnp.full((16,), prefix, jnp.int32), jnp.float32)
        return jnp.where(lanes == j, thrf, thrvec)

    thrvec = lax.fori_loop(0, _COLS_PER_TILE, process_column,
                           jnp.zeros((16,), jnp.float32))
    thrbuf[...] = thrvec
    pltpu.sync_copy(thrbuf, out_hbm.at[wid])


_NTILES = 32
_COLS_PER_TILE = 8


def _sc_select(a1T, kcount):
    U, B = a1T.shape
    mesh = plsc.VectorSubcoreMesh(core_axis_name="c", subcore_axis_name="s")
    sel = pl.kernel(
        functools.partial(_sc_select_body, kcount),
        mesh=mesh,
        compiler_params=pltpu.CompilerParams(needs_layout_passes=False),
        out_type=jax.ShapeDtypeStruct((_NTILES, 16), jnp.float32),
        scratch_types=[
            pltpu.VMEM((B,), jnp.float32),
            pltpu.VMEM((B + 128,), jnp.int32),
            pltpu.VMEM((B + 128,), jnp.int32),
            pltpu.VMEM((_HIST_WORDS,), jnp.int32),
            pltpu.VMEM((16,), jnp.float32),
        ],
    )
    out = sel(a1T)
    return out[:, :_COLS_PER_TILE].reshape(U, 1)


def _decode_body(a1t_ref, w_ref, thr_ref, db_ref, out_ref):
    a = a1t_ref[...]
    am = jnp.where(a >= thr_ref[...], a, 0.0)
    out = lax.dot_general(
        am, w_ref[...], (((0,), (0,)), ((), ())),
        preferred_element_type=jnp.float32,
        precision=lax.Precision.DEFAULT,
    )
    out_ref[...] = out + db_ref[...]


def _decode(a1T, W, thr, db2d):
    U, B = a1T.shape
    D = W.shape[1]
    grid = B // BATCH_BLK
    return pl.pallas_call(
        _decode_body,
        grid=(grid,),
        in_specs=[
            pl.BlockSpec((U, BATCH_BLK), lambda i: (0, i)),
            pl.BlockSpec((U, D), lambda i: (0, 0)),
            pl.BlockSpec((U, 1), lambda i: (0, 0)),
            pl.BlockSpec((1, D), lambda i: (0, 0)),
        ],
        out_specs=pl.BlockSpec((BATCH_BLK, D), lambda i: (i, 0)),
        out_shape=jax.ShapeDtypeStruct((B, D), jnp.float32),
    )(a1T, W, thr, db2d)


def kernel(x, W, b, decoder_bias):
    B = x.shape[0]
    kcount = max(1, int(B * 0.05))
    a1T = _encode(x, W, b.reshape(-1, 1))
    thr = _sc_select(a1T, kcount)
    return _decode(a1T, W, thr, decoder_bias.reshape(1, -1))


# speedup vs baseline: 1.2533x; 1.2533x over previous
import functools

import jax
import jax.numpy as jnp
from jax import lax
from jax.experimental import pallas as pl
from jax.experimental.pallas import tpu as pltpu
from jax.experimental.pallas import tpu_sc as plsc

BATCH_BLK = 2048


def _encode_body(x_ref, w_ref, b_ref, a1t_ref):
    xblk = x_ref[...]
    z = lax.dot_general(
        w_ref[...], xblk, (((1,), (1,)), ((), ())),
        preferred_element_type=jnp.float32,
        precision=lax.Precision.DEFAULT,
    )
    z = z + b_ref[...]
    a1t_ref[...] = jnp.where(z > 0, z, 0.0)


def _encode(x, W, b2d):
    B, D = x.shape
    U = W.shape[0]
    grid = B // BATCH_BLK
    return pl.pallas_call(
        _encode_body,
        grid=(grid,),
        in_specs=[
            pl.BlockSpec((BATCH_BLK, D), lambda i: (i, 0)),
            pl.BlockSpec((U, D), lambda i: (0, 0)),
            pl.BlockSpec((U, 1), lambda i: (0, 0)),
        ],
        out_specs=pl.BlockSpec((U, BATCH_BLK), lambda i: (0, i)),
        out_shape=jax.ShapeDtypeStruct((U, B), jnp.float32),
    )(x, W, b2d)


def _select_body(kcount, a1t_ref, thr_ref):
    a = a1t_ref[...]
    U = a.shape[0]
    kf = jnp.float32(kcount)

    def step(i, t):
        bit = 30 - i
        cand = t | (jnp.int32(1) << bit)
        candf = lax.bitcast_convert_type(cand, jnp.float32)
        cnt = jnp.sum(jnp.where(a >= candf, 1.0, 0.0), axis=1, keepdims=True)
        return jnp.where(cnt >= kf, cand, t)

    t0 = jnp.zeros((U, 1), jnp.int32)
    t = lax.fori_loop(0, 31, step, t0)
    thr_ref[...] = lax.bitcast_convert_type(t, jnp.float32)


def _select(a1T, kcount):
    U = a1T.shape[0]
    return pl.pallas_call(
        functools.partial(_select_body, kcount),
        out_shape=jax.ShapeDtypeStruct((U, 1), jnp.float32),
    )(a1T)


_SUB_PASSES = ((16, 6), (10, 6), (4, 6), (0, 4))
_P1_SHIFT, _P1_BITS = 22, 10
_P1_NB = 1 << (_P1_BITS - 1)
_P1_STRIDE = _P1_NB + 1
_HIST_WORDS = 16 * _P1_STRIDE + 112


def _sc_select_body(kcount, a1t_hbm, out_hbm, colbuf, cand1, cand2, hist,
                    thrbuf, dmasem):
    info = plsc.get_sparse_core_info()
    nc = info.num_cores
    wid = lax.axis_index("s") * nc + lax.axis_index("c")
    lanes = lax.iota(jnp.int32, 16)
    ones = jnp.ones((16,), jnp.int32)
    kc = jnp.int32(kcount)
    B = a1t_hbm.shape[1]
    U = a1t_hbm.shape[0]

    def col_dma(col, base):
        return pltpu.make_async_copy(
            a1t_hbm.at[col], colbuf.at[pl.ds(base, B)], dmasem)

    def zero_hist(nwords):
        def zb(i, _):
            for t in range(8):
                hist[pl.ds((i * 8 + t) * 16, 16)] = jnp.zeros((16,), jnp.int32)
            return 0
        lax.fori_loop(0, (nwords + 127) // 128, zb, 0)

    def group_detail(stride, base, above, r):
        acc = jnp.zeros((16,), jnp.int32)
        for l in range(16):
            acc = acc + hist[pl.ds(l * stride + base, 16)]
        s = lax.rev(jnp.cumsum(lax.rev(acc, (0,))), (0,))
        si = s + above
        se = si - acc
        m = (se < r) & (si >= r)
        bmax = lax.reduce_max(jnp.where(m, base + lanes, jnp.int32(-1)), (0,))
        semax = lax.reduce_max(jnp.where(m, se, jnp.int32(0)), (0,))
        return bmax, semax, lax.reduce_sum(acc, (0,))

    def find_bucket(nb, stride, base_list, r):
        above = jnp.int32(0)
        bstar = jnp.int32(-1)
        rnew = jnp.int32(-1)
        for g in reversed(base_list):
            bmax, semax, gtot = group_detail(stride, g, above, r)
            found = bmax >= 0
            bstar = jnp.where(found, bmax, bstar)
            rnew = jnp.where(found, r - semax, rnew)
            above = above + gtot
        return bstar, rnew

    def find_bucket_2level(stride, r):
        def gt_body(gi, carry):
            gtv0, gtv1 = carry
            acc = jnp.zeros((16,), jnp.int32)
            base = gi * 16
            for l in range(16):
                acc = acc + hist[pl.ds(l * stride + base, 16)]
            totv = jnp.full((16,), lax.reduce_sum(acc, (0,)), jnp.int32)
            gtv0 = jnp.where(lanes == gi, totv, gtv0)
            gtv1 = jnp.where(lanes == gi - 16, totv, gtv1)
            return gtv0, gtv1
        z16 = jnp.zeros((16,), jnp.int32)
        gtv0, gtv1 = lax.fori_loop(0, 32, gt_body, (z16, z16))
        s1 = lax.rev(jnp.cumsum(lax.rev(gtv1, (0,))), (0,))
        s0 = lax.rev(jnp.cumsum(lax.rev(gtv0, (0,))), (0,))
        tot1 = lax.reduce_sum(gtv1, (0,))
        si0 = s0 + tot1
        se0 = si0 - gtv0
        se1 = s1 - gtv1
        m0 = (se0 < r) & (si0 >= r)
        m1 = (se1 < r) & (s1 >= r)
        g0 = lax.reduce_max(jnp.where(m0, lanes, jnp.int32(-1)), (0,))
        g1 = lax.reduce_max(jnp.where(m1, lanes + 16, jnp.int32(-1)), (0,))
        gc = jnp.maximum(g0, g1)
        above = jnp.maximum(
            lax.reduce_max(jnp.where(m0, se0, jnp.int32(0)), (0,)),
            lax.reduce_max(jnp.where(m1, se1, jnp.int32(0)), (0,)))
        bmax, semax, _ = group_detail(stride, gc * 16, above, r)
        return bmax, r - semax

    def process_column(j, thrvec):
        col = wid * jnp.int32(_COLS_PER_TILE) + j
        bufbase = (j & 1) * B
        col_dma(col, bufbase).wait()

        @pl.when(j < _COLS_PER_TILE - 1)
        def _prefetch():
            col_dma(col + 1, B - bufbase).start()

        nvec = B // 16

        zero_hist(16 * _P1_STRIDE)
        lane_base1 = lanes * jnp.int32(_P1_STRIDE)

        def h1(i, _):
            for t in range(8):
                v = colbuf[pl.ds(bufbase + (i * 8 + t) * 16, 16)]
                u = lax.bitcast_convert_type(v, jnp.int32)
                bkt = lax.shift_right_logical(u, jnp.int32(_P1_SHIFT))
                plsc.addupdate_scatter(hist, [lane_base1 + bkt], ones)
            return 0
        lax.fori_loop(0, nvec // 8, h1, 0)
        b1, r = find_bucket_2level(_P1_STRIDE, kc)

        def cp1(i, cnt):
            for t in range(8):
                v = colbuf[pl.ds(bufbase + (i * 8 + t) * 16, 16)]
                u = lax.bitcast_convert_type(v, jnp.int32)
                m = lax.shift_right_logical(u, jnp.int32(_P1_SHIFT)) == b1
                plsc.store_compressed(cand1.at[pl.ds(cnt, 16)], u, mask=m)
                cnt = cnt + plsc.all_reduce_population_count(m)[0]
            return cnt
        n = lax.fori_loop(0, nvec // 8, cp1, jnp.int32(0))
        prefix = b1
        src, dst = cand1, cand2

        for pi, (shift, bits) in enumerate(_SUB_PASSES):
            nb = 1 << bits
            stride = nb + 1
            zero_hist(16 * stride)
            lane_base = lanes * jnp.int32(stride)
            shift_c = jnp.int32(shift)
            mask_c = jnp.int32(nb - 1)
            nvr = lax.shift_right_logical(n + jnp.int32(15), 4)

            def hb(i, _):
                base_i = i * 16
                u = src[pl.ds(base_i, 16)]
                valid = (base_i + lanes) < n
                bkt = lax.shift_right_logical(u, shift_c) & mask_c
                plsc.addupdate_scatter(hist, [lane_base + bkt], ones,
                                       mask=valid)
                return 0
            lax.fori_loop(0, nvr, hb, 0)
            bstar, r = find_bucket(nb, stride,
                                   [g * 16 for g in range(nb // 16)], r)
            prefix = lax.shift_left(prefix, jnp.int32(bits)) | bstar

            if pi != len(_SUB_PASSES) - 1:
                def cpb(i, cnt):
                    base_i = i * 16
                    u = src[pl.ds(base_i, 16)]
                    valid = (base_i + lanes) < n
                    m = (lax.shift_right_logical(u, shift_c) == prefix) & valid
                    plsc.store_compressed(dst.at[pl.ds(cnt, 16)], u, mask=m)
                    return cnt + plsc.all_reduce_population_count(m)[0]
                n = lax.fori_loop(0, nvr, cpb, jnp.int32(0))
                src, dst = dst, src

        thrf = lax.bitcast_convert_type(jnp.full((16,), prefix, jnp.int32), jnp.float32)
        return jnp.where(lanes == j, thrf, thrvec)

    col_dma(wid * jnp.int32(_COLS_PER_TILE), 0).start()
    thrvec = lax.fori_loop(0, _COLS_PER_TILE, process_column,
                           jnp.zeros((16,), jnp.float32))
    thrbuf[...] = thrvec
    pltpu.sync_copy(thrbuf, out_hbm.at[wid])


_NTILES = 32
_COLS_PER_TILE = 8


def _sc_select(a1T, kcount):
    U, B = a1T.shape
    mesh = plsc.VectorSubcoreMesh(core_axis_name="c", subcore_axis_name="s")
    sel = pl.kernel(
        functools.partial(_sc_select_body, kcount),
        mesh=mesh,
        compiler_params=pltpu.CompilerParams(needs_layout_passes=False),
        out_type=jax.ShapeDtypeStruct((_NTILES, 16), jnp.float32),
        scratch_types=[
            pltpu.VMEM((2 * B,), jnp.float32),
            pltpu.VMEM((B + 128,), jnp.int32),
            pltpu.VMEM((B + 128,), jnp.int32),
            pltpu.VMEM((_HIST_WORDS,), jnp.int32),
            pltpu.VMEM((16,), jnp.float32),
            pltpu.SemaphoreType.DMA,
        ],
    )
    out = sel(a1T)
    return out[:, :_COLS_PER_TILE].reshape(U, 1)


def _decode_body(a1t_ref, w_ref, thr_ref, db_ref, out_ref):
    a = a1t_ref[...]
    am = jnp.where(a >= thr_ref[...], a, 0.0)
    out = lax.dot_general(
        am, w_ref[...], (((0,), (0,)), ((), ())),
        preferred_element_type=jnp.float32,
        precision=lax.Precision.DEFAULT,
    )
    out_ref[...] = out + db_ref[...]


def _decode(a1T, W, thr, db2d):
    U, B = a1T.shape
    D = W.shape[1]
    grid = B // BATCH_BLK
    return pl.pallas_call(
        _decode_body,
        grid=(grid,),
        in_specs=[
            pl.BlockSpec((U, BATCH_BLK), lambda i: (0, i)),
            pl.BlockSpec((U, D), lambda i: (0, 0)),
            pl.BlockSpec((U, 1), lambda i: (0, 0)),
            pl.BlockSpec((1, D), lambda i: (0, 0)),
        ],
        out_specs=pl.BlockSpec((BATCH_BLK, D), lambda i: (i, 0)),
        out_shape=jax.ShapeDtypeStruct((B, D), jnp.float32),
    )(a1T, W, thr, db2d)


def kernel(x, W, b, decoder_bias):
    B = x.shape[0]
    kcount = max(1, int(B * 0.05))
    a1T = _encode(x, W, b.reshape(-1, 1))
    thr = _sc_select(a1T, kcount)
    return _decode(a1T, W, thr, decoder_bias.reshape(1, -1))


# speedup vs baseline: 3.2766x; 2.6143x over previous
import functools

import jax
import jax.numpy as jnp
from jax import lax
from jax.experimental import pallas as pl
from jax.experimental.pallas import tpu as pltpu
from jax.experimental.pallas import tpu_sc as plsc

BATCH_BLK = 2048


def _encode_body(x_ref, w_ref, b_ref, a1t_ref):
    xblk = x_ref[...]
    z = lax.dot_general(
        w_ref[...], xblk, (((1,), (1,)), ((), ())),
        preferred_element_type=jnp.float32,
        precision=lax.Precision.DEFAULT,
    )
    z = z + b_ref[...]
    a1t_ref[...] = jnp.where(z > 0, z, 0.0)


def _encode(x, W, b2d):
    B, D = x.shape
    U = W.shape[0]
    grid = B // BATCH_BLK
    return pl.pallas_call(
        _encode_body,
        grid=(grid,),
        in_specs=[
            pl.BlockSpec((BATCH_BLK, D), lambda i: (i, 0)),
            pl.BlockSpec((U, D), lambda i: (0, 0)),
            pl.BlockSpec((U, 1), lambda i: (0, 0)),
        ],
        out_specs=pl.BlockSpec((U, BATCH_BLK), lambda i: (0, i)),
        out_shape=jax.ShapeDtypeStruct((U, B), jnp.float32),
    )(x, W, b2d)


def _select_body(kcount, a1t_ref, thr_ref):
    a = a1t_ref[...]
    U = a.shape[0]
    kf = jnp.float32(kcount)

    def step(i, t):
        bit = 30 - i
        cand = t | (jnp.int32(1) << bit)
        candf = lax.bitcast_convert_type(cand, jnp.float32)
        cnt = jnp.sum(jnp.where(a >= candf, 1.0, 0.0), axis=1, keepdims=True)
        return jnp.where(cnt >= kf, cand, t)

    t0 = jnp.zeros((U, 1), jnp.int32)
    t = lax.fori_loop(0, 31, step, t0)
    thr_ref[...] = lax.bitcast_convert_type(t, jnp.float32)


def _select(a1T, kcount):
    U = a1T.shape[0]
    return pl.pallas_call(
        functools.partial(_select_body, kcount),
        out_shape=jax.ShapeDtypeStruct((U, 1), jnp.float32),
    )(a1T)


_SUB_PASSES = ((16, 6), (10, 6), (4, 6), (0, 4))
_P1_SHIFT, _P1_BITS = 22, 10
_P1_NB = 1 << (_P1_BITS - 1)
_P1_STRIDE = _P1_NB + 1
_HIST_WORDS = 16 * _P1_STRIDE + 112


def _sc_select_body(kcount, a1t_hbm, out_hbm, colbuf, cand1, cand2, hist,
                    thrbuf, dmasem):
    info = plsc.get_sparse_core_info()
    nc = info.num_cores
    wid = lax.axis_index("s") * nc + lax.axis_index("c")
    lanes = lax.iota(jnp.int32, 16)
    ones = jnp.ones((16,), jnp.int32)
    kc = jnp.int32(kcount)
    B = a1t_hbm.shape[1]
    U = a1t_hbm.shape[0]

    def col_dma(col, base):
        return pltpu.make_async_copy(
            a1t_hbm.at[col], colbuf.at[pl.ds(base, B)], dmasem)

    def zero_hist(nwords):
        def zb(i):
            hist[pl.ds(i * 16, 16)] = jnp.zeros((16,), jnp.int32)
        plsc.parallel_loop(0, nwords // 16, 1, unroll=8)(zb)

    def group_detail(stride, base, above, r):
        acc = jnp.zeros((16,), jnp.int32)
        for l in range(16):
            acc = acc + hist[pl.ds(l * stride + base, 16)]
        s = lax.rev(jnp.cumsum(lax.rev(acc, (0,))), (0,))
        si = s + above
        se = si - acc
        m = (se < r) & (si >= r)
        bmax = lax.reduce_max(jnp.where(m, base + lanes, jnp.int32(-1)), (0,))
        semax = lax.reduce_max(jnp.where(m, se, jnp.int32(0)), (0,))
        return bmax, semax, lax.reduce_sum(acc, (0,))

    def find_bucket(nb, stride, base_list, r):
        above = jnp.int32(0)
        bstar = jnp.int32(-1)
        rnew = jnp.int32(-1)
        for g in reversed(base_list):
            bmax, semax, gtot = group_detail(stride, g, above, r)
            found = bmax >= 0
            bstar = jnp.where(found, bmax, bstar)
            rnew = jnp.where(found, r - semax, rnew)
            above = above + gtot
        return bstar, rnew

    def find_bucket_2level(stride, r):
        def gt_body(gi, carry):
            gtv0, gtv1 = carry
            acc = jnp.zeros((16,), jnp.int32)
            base = gi * 16
            for l in range(16):
                acc = acc + hist[pl.ds(l * stride + base, 16)]
            totv = jnp.full((16,), lax.reduce_sum(acc, (0,)), jnp.int32)
            gtv0 = jnp.where(lanes == gi, totv, gtv0)
            gtv1 = jnp.where(lanes == gi - 16, totv, gtv1)
            return gtv0, gtv1
        z16 = jnp.zeros((16,), jnp.int32)
        gtv0, gtv1 = plsc.parallel_loop(
            0, 32, 1, unroll=2, carry=(z16, z16))(gt_body)
        s1 = lax.rev(jnp.cumsum(lax.rev(gtv1, (0,))), (0,))
        s0 = lax.rev(jnp.cumsum(lax.rev(gtv0, (0,))), (0,))
        tot1 = lax.reduce_sum(gtv1, (0,))
        si0 = s0 + tot1
        se0 = si0 - gtv0
        se1 = s1 - gtv1
        m0 = (se0 < r) & (si0 >= r)
        m1 = (se1 < r) & (s1 >= r)
        g0 = lax.reduce_max(jnp.where(m0, lanes, jnp.int32(-1)), (0,))
        g1 = lax.reduce_max(jnp.where(m1, lanes + 16, jnp.int32(-1)), (0,))
        gc = jnp.maximum(g0, g1)
        above = jnp.maximum(
            lax.reduce_max(jnp.where(m0, se0, jnp.int32(0)), (0,)),
            lax.reduce_max(jnp.where(m1, se1, jnp.int32(0)), (0,)))
        bmax, semax, _ = group_detail(stride, gc * 16, above, r)
        return bmax, r - semax

    def process_column(j, thrvec):
        col = wid * jnp.int32(_COLS_PER_TILE) + j
        bufbase = (j & 1) * B
        col_dma(col, bufbase).wait()

        @pl.when(j < _COLS_PER_TILE - 1)
        def _prefetch():
            col_dma(col + 1, B - bufbase).start()

        nvec = B // 16

        zero_hist(16 * _P1_STRIDE)
        lane_base1 = lanes * jnp.int32(_P1_STRIDE)

        def h1(i):
            v = colbuf[pl.ds(bufbase + i * 16, 16)]
            u = lax.bitcast_convert_type(v, jnp.int32)
            bkt = lax.shift_right_logical(u, jnp.int32(_P1_SHIFT))
            plsc.addupdate_scatter(hist, [lane_base1 + bkt], ones)
        plsc.parallel_loop(0, nvec, 1, unroll=8)(h1)
        b1, r = find_bucket_2level(_P1_STRIDE, kc)

        def cp1(i, cnt):
            v = colbuf[pl.ds(bufbase + i * 16, 16)]
            u = lax.bitcast_convert_type(v, jnp.int32)
            m = lax.shift_right_logical(u, jnp.int32(_P1_SHIFT)) == b1
            plsc.store_compressed(cand1.at[pl.ds(cnt, 16)], u, mask=m)
            return cnt + plsc.all_reduce_population_count(m)[0]
        n = plsc.parallel_loop(0, nvec, 1, unroll=8,
                               carry=jnp.int32(0))(cp1)
        prefix = b1
        src, dst = cand1, cand2

        for pi, (shift, bits) in enumerate(_SUB_PASSES):
            nb = 1 << bits
            stride = nb + 1
            zero_hist(16 * stride)
            lane_base = lanes * jnp.int32(stride)
            shift_c = jnp.int32(shift)
            mask_c = jnp.int32(nb - 1)
            nvr = lax.shift_right_logical(n + jnp.int32(15), 4)

            def hb(i):
                base_i = i * 16
                u = src[pl.ds(base_i, 16)]
                valid = (base_i + lanes) < n
                bkt = lax.shift_right_logical(u, shift_c) & mask_c
                plsc.addupdate_scatter(hist, [lane_base + bkt], ones,
                                       mask=valid)
            plsc.parallel_loop(0, nvr, 1, unroll=2)(hb)
            bstar, r = find_bucket(nb, stride,
                                   [g * 16 for g in range(nb // 16)], r)
            prefix = lax.shift_left(prefix, jnp.int32(bits)) | bstar

            if pi != len(_SUB_PASSES) - 1:
                def cpb(i, cnt):
                    base_i = i * 16
                    u = src[pl.ds(base_i, 16)]
                    valid = (base_i + lanes) < n
                    m = (lax.shift_right_logical(u, shift_c) == prefix) & valid
                    plsc.store_compressed(dst.at[pl.ds(cnt, 16)], u, mask=m)
                    return cnt + plsc.all_reduce_population_count(m)[0]
                n = plsc.parallel_loop(0, nvr, 1, unroll=2,
                                       carry=jnp.int32(0))(cpb)
                src, dst = dst, src

        thrf = lax.bitcast_convert_type(jnp.full((16,), prefix, jnp.int32), jnp.float32)
        return jnp.where(lanes == j, thrf, thrvec)

    col_dma(wid * jnp.int32(_COLS_PER_TILE), 0).start()
    thrvec = lax.fori_loop(0, _COLS_PER_TILE, process_column,
                           jnp.zeros((16,), jnp.float32))
    thrbuf[...] = thrvec
    pltpu.sync_copy(thrbuf, out_hbm.at[wid])


_NTILES = 32
_COLS_PER_TILE = 8


def _sc_select(a1T, kcount):
    U, B = a1T.shape
    mesh = plsc.VectorSubcoreMesh(core_axis_name="c", subcore_axis_name="s")
    sel = pl.kernel(
        functools.partial(_sc_select_body, kcount),
        mesh=mesh,
        compiler_params=pltpu.CompilerParams(needs_layout_passes=False),
        out_type=jax.ShapeDtypeStruct((_NTILES, 16), jnp.float32),
        scratch_types=[
            pltpu.VMEM((2 * B,), jnp.float32),
            pltpu.VMEM((B + 128,), jnp.int32),
            pltpu.VMEM((B + 128,), jnp.int32),
            pltpu.VMEM((_HIST_WORDS,), jnp.int32),
            pltpu.VMEM((16,), jnp.float32),
            pltpu.SemaphoreType.DMA,
        ],
    )
    out = sel(a1T)
    return out[:, :_COLS_PER_TILE].reshape(U, 1)


def _decode_body(a1t_ref, w_ref, thr_ref, db_ref, out_ref):
    a = a1t_ref[...]
    am = jnp.where(a >= thr_ref[...], a, 0.0)
    out = lax.dot_general(
        am, w_ref[...], (((0,), (0,)), ((), ())),
        preferred_element_type=jnp.float32,
        precision=lax.Precision.DEFAULT,
    )
    out_ref[...] = out + db_ref[...]


def _decode(a1T, W, thr, db2d):
    U, B = a1T.shape
    D = W.shape[1]
    grid = B // BATCH_BLK
    return pl.pallas_call(
        _decode_body,
        grid=(grid,),
        in_specs=[
            pl.BlockSpec((U, BATCH_BLK), lambda i: (0, i)),
            pl.BlockSpec((U, D), lambda i: (0, 0)),
            pl.BlockSpec((U, 1), lambda i: (0, 0)),
            pl.BlockSpec((1, D), lambda i: (0, 0)),
        ],
        out_specs=pl.BlockSpec((BATCH_BLK, D), lambda i: (i, 0)),
        out_shape=jax.ShapeDtypeStruct((B, D), jnp.float32),
    )(a1T, W, thr, db2d)


def kernel(x, W, b, decoder_bias):
    B = x.shape[0]
    kcount = max(1, int(B * 0.05))
    a1T = _encode(x, W, b.reshape(-1, 1))
    thr = _sc_select(a1T, kcount)
    return _decode(a1T, W, thr, decoder_bias.reshape(1, -1))


# speedup vs baseline: 3.2819x; 1.0016x over previous
import functools

import jax
import jax.numpy as jnp
from jax import lax
from jax.experimental import pallas as pl
from jax.experimental.pallas import tpu as pltpu
from jax.experimental.pallas import tpu_sc as plsc

BATCH_BLK = 2048


def _encode_body(x_ref, w_ref, b_ref, a1t_ref):
    xblk = x_ref[...]
    z = lax.dot_general(
        w_ref[...], xblk, (((1,), (1,)), ((), ())),
        preferred_element_type=jnp.float32,
        precision=lax.Precision.DEFAULT,
    )
    z = z + b_ref[...]
    a1t_ref[...] = jnp.where(z > 0, z, 0.0)


def _encode(x, W, b2d):
    B, D = x.shape
    U = W.shape[0]
    grid = B // BATCH_BLK
    return pl.pallas_call(
        _encode_body,
        grid=(grid,),
        in_specs=[
            pl.BlockSpec((BATCH_BLK, D), lambda i: (i, 0)),
            pl.BlockSpec((U, D), lambda i: (0, 0)),
            pl.BlockSpec((U, 1), lambda i: (0, 0)),
        ],
        out_specs=pl.BlockSpec((U, BATCH_BLK), lambda i: (0, i)),
        out_shape=jax.ShapeDtypeStruct((U, B), jnp.float32),
    )(x, W, b2d)


_SUB_PASSES = ((16, 6), (10, 6), (4, 6), (0, 4))
_P1_SHIFT, _P1_BITS = 22, 10
_P1_NB = 1 << (_P1_BITS - 1)
_P1_STRIDE = _P1_NB + 1
_HIST_WORDS = 16 * _P1_STRIDE + 112


def _sc_select_body(kcount, a1t_hbm, out_hbm, colbuf, cand1, cand2, hist,
                    thrbuf, dmasem):
    info = plsc.get_sparse_core_info()
    nc = info.num_cores
    wid = lax.axis_index("s") * nc + lax.axis_index("c")
    lanes = lax.iota(jnp.int32, 16)
    ones = jnp.ones((16,), jnp.int32)
    kc = jnp.int32(kcount)
    B = a1t_hbm.shape[1]
    U = a1t_hbm.shape[0]

    def col_dma(col, base):
        return pltpu.make_async_copy(
            a1t_hbm.at[col], colbuf.at[pl.ds(base, B)], dmasem)

    def zero_hist(nwords):
        def zb(i):
            hist[pl.ds(i * 16, 16)] = jnp.zeros((16,), jnp.int32)
        plsc.parallel_loop(0, nwords // 16, 1, unroll=8)(zb)

    def group_detail(stride, base, above, r):
        acc = jnp.zeros((16,), jnp.int32)
        for l in range(16):
            acc = acc + hist[pl.ds(l * stride + base, 16)]
        s = lax.rev(jnp.cumsum(lax.rev(acc, (0,))), (0,))
        si = s + above
        se = si - acc
        m = (se < r) & (si >= r)
        bmax = lax.reduce_max(jnp.where(m, base + lanes, jnp.int32(-1)), (0,))
        semax = lax.reduce_max(jnp.where(m, se, jnp.int32(0)), (0,))
        return bmax, semax, lax.reduce_sum(acc, (0,))

    def find_bucket(nb, stride, base_list, r):
        above = jnp.int32(0)
        bstar = jnp.int32(-1)
        rnew = jnp.int32(-1)
        for g in reversed(base_list):
            bmax, semax, gtot = group_detail(stride, g, above, r)
            found = bmax >= 0
            bstar = jnp.where(found, bmax, bstar)
            rnew = jnp.where(found, r - semax, rnew)
            above = above + gtot
        return bstar, rnew

    def find_bucket_2level(stride, r):
        def gt_body(gi, carry):
            gtv0, gtv1 = carry
            acc = jnp.zeros((16,), jnp.int32)
            base = gi * 16
            for l in range(16):
                acc = acc + hist[pl.ds(l * stride + base, 16)]
            totv = jnp.full((16,), lax.reduce_sum(acc, (0,)), jnp.int32)
            gtv0 = jnp.where(lanes == gi, totv, gtv0)
            gtv1 = jnp.where(lanes == gi - 16, totv, gtv1)
            return gtv0, gtv1
        z16 = jnp.zeros((16,), jnp.int32)
        gtv0, gtv1 = plsc.parallel_loop(
            0, 32, 1, unroll=2, carry=(z16, z16))(gt_body)
        s1 = lax.rev(jnp.cumsum(lax.rev(gtv1, (0,))), (0,))
        s0 = lax.rev(jnp.cumsum(lax.rev(gtv0, (0,))), (0,))
        tot1 = lax.reduce_sum(gtv1, (0,))
        si0 = s0 + tot1
        se0 = si0 - gtv0
        se1 = s1 - gtv1
        m0 = (se0 < r) & (si0 >= r)
        m1 = (se1 < r) & (s1 >= r)
        g0 = lax.reduce_max(jnp.where(m0, lanes, jnp.int32(-1)), (0,))
        g1 = lax.reduce_max(jnp.where(m1, lanes + 16, jnp.int32(-1)), (0,))
        gc = jnp.maximum(g0, g1)
        above = jnp.maximum(
            lax.reduce_max(jnp.where(m0, se0, jnp.int32(0)), (0,)),
            lax.reduce_max(jnp.where(m1, se1, jnp.int32(0)), (0,)))
        bmax, semax, _ = group_detail(stride, gc * 16, above, r)
        return bmax, r - semax

    def process_column(j, thrvec):
        col = wid * jnp.int32(_COLS_PER_TILE) + j
        bufbase = (j & 1) * B
        col_dma(col, bufbase).wait()

        @pl.when(j < _COLS_PER_TILE - 1)
        def _prefetch():
            col_dma(col + 1, B - bufbase).start()

        nvec = B // 16

        zero_hist(16 * _P1_STRIDE)
        lane_base1 = lanes * jnp.int32(_P1_STRIDE)

        def h1(i):
            v = colbuf[pl.ds(bufbase + i * 16, 16)]
            u = lax.bitcast_convert_type(v, jnp.int32)
            bkt = lax.shift_right_logical(u, jnp.int32(_P1_SHIFT))
            plsc.addupdate_scatter(hist, [lane_base1 + bkt], ones)
        plsc.parallel_loop(0, nvec, 1, unroll=8)(h1)
        b1, r = find_bucket_2level(_P1_STRIDE, kc)

        def cp1(i, cnt):
            v = colbuf[pl.ds(bufbase + i * 16, 16)]
            u = lax.bitcast_convert_type(v, jnp.int32)
            m = lax.shift_right_logical(u, jnp.int32(_P1_SHIFT)) == b1
            plsc.store_compressed(cand1.at[pl.ds(cnt, 16)], u, mask=m)
            return cnt + plsc.all_reduce_population_count(m)[0]
        n = plsc.parallel_loop(0, nvec, 1, unroll=8,
                               carry=jnp.int32(0))(cp1)
        prefix = b1
        src, dst = cand1, cand2

        for pi, (shift, bits) in enumerate(_SUB_PASSES):
            nb = 1 << bits
            stride = nb + 1
            zero_hist(16 * stride)
            lane_base = lanes * jnp.int32(stride)
            shift_c = jnp.int32(shift)
            mask_c = jnp.int32(nb - 1)
            nvr = lax.shift_right_logical(n + jnp.int32(15), 4)

            def hb(i):
                base_i = i * 16
                u = src[pl.ds(base_i, 16)]
                valid = (base_i + lanes) < n
                bkt = lax.shift_right_logical(u, shift_c) & mask_c
                plsc.addupdate_scatter(hist, [lane_base + bkt], ones,
                                       mask=valid)
            plsc.parallel_loop(0, nvr, 1, unroll=2)(hb)
            bstar, r = find_bucket(nb, stride,
                                   [g * 16 for g in range(nb // 16)], r)
            prefix = lax.shift_left(prefix, jnp.int32(bits)) | bstar

            if pi != len(_SUB_PASSES) - 1:
                def cpb(i, cnt):
                    base_i = i * 16
                    u = src[pl.ds(base_i, 16)]
                    valid = (base_i + lanes) < n
                    m = (lax.shift_right_logical(u, shift_c) == prefix) & valid
                    plsc.store_compressed(dst.at[pl.ds(cnt, 16)], u, mask=m)
                    return cnt + plsc.all_reduce_population_count(m)[0]
                n = plsc.parallel_loop(0, nvr, 1, unroll=2,
                                       carry=jnp.int32(0))(cpb)
                src, dst = dst, src

        thrf = lax.bitcast_convert_type(jnp.full((16,), prefix, jnp.int32), jnp.float32)
        return jnp.where(lanes == j, thrf, thrvec)

    col_dma(wid * jnp.int32(_COLS_PER_TILE), 0).start()
    thrvec = lax.fori_loop(0, _COLS_PER_TILE, process_column,
                           jnp.zeros((16,), jnp.float32))
    thrbuf[...] = thrvec
    pltpu.sync_copy(thrbuf, out_hbm.at[wid])


_NTILES = 32
_COLS_PER_TILE = 8


def _sc_select(a1T, kcount):
    U, B = a1T.shape
    mesh = plsc.VectorSubcoreMesh(core_axis_name="c", subcore_axis_name="s")
    sel = pl.kernel(
        functools.partial(_sc_select_body, kcount),
        mesh=mesh,
        compiler_params=pltpu.CompilerParams(needs_layout_passes=False),
        out_type=jax.ShapeDtypeStruct((_NTILES, 16), jnp.float32),
        scratch_types=[
            pltpu.VMEM((2 * B,), jnp.float32),
            pltpu.VMEM((B + 128,), jnp.int32),
            pltpu.VMEM((B + 128,), jnp.int32),
            pltpu.VMEM((_HIST_WORDS,), jnp.int32),
            pltpu.VMEM((16,), jnp.float32),
            pltpu.SemaphoreType.DMA,
        ],
    )
    out = sel(a1T)
    return out[:, :_COLS_PER_TILE].reshape(U, 1)


def _decode_body(a1t_ref, w_ref, thr_ref, db_ref, out_ref):
    a = a1t_ref[...]
    am = jnp.where(a >= thr_ref[...], a, 0.0)
    out = lax.dot_general(
        am, w_ref[...], (((0,), (0,)), ((), ())),
        preferred_element_type=jnp.float32,
        precision=lax.Precision.DEFAULT,
    )
    out_ref[...] = out + db_ref[...]


def _decode(a1T, W, thr, db2d):
    U, B = a1T.shape
    D = W.shape[1]
    grid = B // BATCH_BLK
    return pl.pallas_call(
        _decode_body,
        grid=(grid,),
        in_specs=[
            pl.BlockSpec((U, BATCH_BLK), lambda i: (0, i)),
            pl.BlockSpec((U, D), lambda i: (0, 0)),
            pl.BlockSpec((U, 1), lambda i: (0, 0)),
            pl.BlockSpec((1, D), lambda i: (0, 0)),
        ],
        out_specs=pl.BlockSpec((BATCH_BLK, D), lambda i: (i, 0)),
        out_shape=jax.ShapeDtypeStruct((B, D), jnp.float32),
    )(a1T, W, thr, db2d)


def kernel(x, W, b, decoder_bias):
    B = x.shape[0]
    kcount = max(1, int(B * 0.05))
    a1T = _encode(x, W, b.reshape(-1, 1))
    thr = _sc_select(a1T, kcount)
    return _decode(a1T, W, thr, decoder_bias.reshape(1, -1))
